# baseline (speedup 1.0000x reference)
import jax
import jax.numpy as jnp
import numpy as np
from jax import lax
from jax.experimental import pallas as pl
from jax.experimental.pallas import tpu as pltpu
from jax.experimental.pallas import tpu_sc as plsc

N, E, D, EA, FCI, FCH = 10000, 320000, 128, 4, 8, 64
NC, NS = 2, 16
NW = NC * NS
C = 128
TE = 4000
TN = 2000
GKG = 3
GKS = 3

f32 = jnp.float32
u32 = jnp.uint32
D2 = D // 2


def _pack_bf16(x):
    xu = jax.lax.bitcast_convert_type(x, u32)
    r = (xu + np.uint32(0x8000)) & np.uint32(0xFFFF0000)
    hi = r[:, :D2]
    lo = r[:, D2:] >> np.uint32(16)
    return jax.lax.bitcast_convert_type(hi | lo, f32)


def _unpack_bf16(p):
    pu = jax.lax.bitcast_convert_type(p, u32)
    a = jax.lax.bitcast_convert_type(pu & np.uint32(0xFFFF0000), f32)
    b = jax.lax.bitcast_convert_type(pu << np.uint32(16), f32)
    return a, b


def _sc_mesh():
    return plsc.VectorSubcoreMesh(core_axis_name="c", subcore_axis_name="s",
                                  num_cores=NC, num_subcores=NS)


def _sc_gather(nf, src, e0, ne):
    nch = ne // C
    cpw = nch // NW
    rem = nch - cpw * NW
    nss = cpw // (2 * GKG)
    tail = cpw - nss * 2 * GKG

    def body(nf_hbm, src_hbm, out_hbm, idx_v, rows_v, semg, sems_a, sems_b):
        wid = lax.axis_index("s") * NC + lax.axis_index("c")
        base = wid * (cpw * C)
        pltpu.sync_copy(src_hbm.at[pl.ds(e0 + base, cpw * C)], idx_v)

        def buf(k):
            return rows_v.at[pl.ds(k * C, C)]

        def gather(lc, k):
            pltpu.async_copy(nf_hbm.at[idx_v.at[pl.ds(lc * C, C)]],
                             buf(k), semg)

        def gdrain(k):
            pltpu.make_async_copy(nf_hbm.at[pl.ds(0, C)], buf(k), semg).wait()

        def store(lc, k, sem):
            pltpu.async_copy(buf(k), out_hbm.at[pl.ds(base + lc * C, C)], sem)

        def sdrain(k, sem):
            pltpu.make_async_copy(buf(k), out_hbm.at[pl.ds(base, C)],
                                  sem).wait()

        def half(s, lc0, k0, sem):
            @pl.when(s > 0)
            def _():
                for k in range(GKG):
                    sdrain(k0 + k, sem)
            for k in range(GKG):
                gather(lc0 + k, k0 + k)
            for k in range(GKG):
                gdrain(k0 + k)
            for k in range(GKG):
                store(lc0 + k, k0 + k, sem)

        def step(s, carry):
            lc0 = s * 2 * GKG
            half(s, lc0, 0, sems_a)
            half(s, lc0 + GKG, GKG, sems_b)
            return carry

        lax.fori_loop(0, nss, step, 0)
        for k in range(GKG):
            sdrain(k, sems_a)
            sdrain(GKG + k, sems_b)
        for t in range(tail):
            lc = nss * 2 * GKG + t
            gather(lc, 0)
            gdrain(0)
            pltpu.sync_copy(buf(0), out_hbm.at[pl.ds(base + lc * C, C)])

        if rem:
            @pl.when(wid < rem)
            def _():
                off = (NW * cpw + wid) * C
                pltpu.sync_copy(src_hbm.at[pl.ds(e0 + off, C)],
                                idx_v.at[pl.ds(0, C)])
                gather(0, 0)
                gdrain(0)
                pltpu.sync_copy(buf(0), out_hbm.at[pl.ds(off, C)])

    return pl.kernel(
        body,
        out_type=jax.ShapeDtypeStruct((ne, D2), f32),
        mesh=_sc_mesh(),
        compiler_params=pltpu.CompilerParams(use_tc_tiling_on_sc=False),
        scratch_types=[
            pltpu.VMEM((cpw * C,), jnp.int32),
            pltpu.VMEM((2 * GKG * C, D2), f32),
            pltpu.SemaphoreType.DMA,
            pltpu.SemaphoreType.DMA,
            pltpu.SemaphoreType.DMA,
        ],
    )(nf, src)


def _sc_scatter(ef, dst, zeros_nd, e0, ne):
    rps = 1000
    nch = ne // C
    cpw = nch // NW
    rem = nch - cpw * NW
    nss = cpw // GKS
    tail = cpw - nss * GKS

    def body(ef_hbm, dst_hbm, z_hbm, agg_hbm, rows_v,
             i0, i1, i2, agg_sp, seml, semi, semc0, semc1, semc2):
        ibufs = (i0, i1, i2)
        semcs = (semc0, semc1, semc2)
        cid = lax.axis_index("c")
        sid = lax.axis_index("s")
        wid = sid * NC + cid

        @pl.when(sid < N // rps)
        def _():
            pltpu.sync_copy(z_hbm.at[pl.ds(sid * rps, rps)],
                            agg_sp.at[pl.ds(sid * rps, rps)])

        plsc.subcore_barrier()
        base = wid * (cpw * C)

        def buf(k):
            return rows_v.at[pl.ds(k * C, C)]

        def load(lc, k):
            pltpu.async_copy(ef_hbm.at[pl.ds(base + lc * C, C)], buf(k), seml)

        def ldrain(k):
            pltpu.make_async_copy(ef_hbm.at[pl.ds(0, C)], buf(k), seml).wait()

        def scat(k):
            pltpu.async_copy(buf(k), agg_sp.at[ibufs[k]], semcs[k], add=True)

        def cdrain(k):
            pltpu.make_async_copy(buf(k), agg_sp.at[pl.ds(0, C)],
                                  semcs[k]).wait()

        def step(s, carry):
            lc0 = s * GKS
            for k in range(GKS):
                @pl.when(s > 0)
                def _(k=k):
                    cdrain(k)
                pltpu.async_copy(dst_hbm.at[pl.ds(e0 + base + (lc0 + k) * C,
                                                  C)],
                                 ibufs[k], semi)
                load(lc0 + k, k)
            for k in range(GKS):
                pltpu.make_async_copy(dst_hbm.at[pl.ds(0, C)],
                                      ibufs[k], semi).wait()
                ldrain(k)
                scat(k)
            return carry

        lax.fori_loop(0, nss, step, 0)
        for k in range(GKS):
            cdrain(k)
        for t in range(tail):
            lc = nss * GKS + t
            pltpu.sync_copy(dst_hbm.at[pl.ds(e0 + base + lc * C, C)], i0)
            pltpu.sync_copy(ef_hbm.at[pl.ds(base + lc * C, C)], buf(0))
            pltpu.sync_copy(buf(0), agg_sp.at[i0], add=True)

        if rem:
            @pl.when(wid < rem)
            def _():
                off = (NW * cpw + wid) * C
                pltpu.sync_copy(dst_hbm.at[pl.ds(e0 + off, C)], i0)
                pltpu.sync_copy(ef_hbm.at[pl.ds(off, C)], buf(0))
                pltpu.sync_copy(buf(0), agg_sp.at[i0], add=True)

        plsc.subcore_barrier()

        @pl.when(sid < N // rps)
        def _():
            pltpu.sync_copy(agg_sp.at[pl.ds(sid * rps, rps)],
                            agg_hbm.at[cid, pl.ds(sid * rps, rps)])

    return pl.kernel(
        body,
        out_type=jax.ShapeDtypeStruct((NC, N, D), f32),
        mesh=_sc_mesh(),
        scratch_types=[
            pltpu.VMEM((GKS * C, D), f32),
            pltpu.VMEM((C,), jnp.int32),
            pltpu.VMEM((C,), jnp.int32),
            pltpu.VMEM((C,), jnp.int32),
            pltpu.VMEM_SHARED((N, D), f32),
            pltpu.SemaphoreType.DMA,
            pltpu.SemaphoreType.DMA,
            pltpu.SemaphoreType.DMA,
            pltpu.SemaphoreType.DMA,
            pltpu.SemaphoreType.DMA,
        ],
    )(ef, dst, zeros_nd)


def _dot(a, b):
    return jnp.dot(a, b, preferred_element_type=f32)


_SIN_C = (9.9997941e-01, -1.6662442e-01, 8.3089975e-03,
          -1.9265186e-04, 2.1479674e-06)


def _fast_sin(x):
    two_pi = np.float32(2.0 * np.pi)
    q = jnp.round(x * np.float32(1.0 / (2.0 * np.pi)))
    r = x - q * two_pi
    r2 = r * r
    acc = jnp.full_like(x, np.float32(_SIN_C[4]))
    for k in (3, 2, 1, 0):
        acc = acc * r2 + np.float32(_SIN_C[k])
    return r * acc


def _edge_pass(es, ea, g, wfc1, w2, wsce, q64, q, r, ob, ea_ob):
    ne = g.shape[0]

    def body(es_ref, ea_ref, g_ref, wfc1_ref, w2_ref, wsce_ref, q64_ref,
             q_ref, r_ref, ef_ref, eaout_ref):
        h = _fast_sin(_dot(es_ref[...], wfc1_ref[...]))
        ea_t = ea_ref[...]
        eaexp64 = _dot(ea_t, q64_ref[...])
        h4 = (jnp.concatenate([h] * EA, axis=1) * eaexp64).astype(jnp.bfloat16)
        wv = _dot(h4, w2_ref[...])
        ga, gb = _unpack_bf16(g_ref[...])
        ef_a = ga * wv[:, :D2]
        ef_b = gb * wv[:, D2:]
        ef_ref[:, :D2] = ef_a
        ef_ref[:, D2:] = ef_b
        ws = wsce_ref[...]
        t = _dot(ef_a, ws[:D2]) + _dot(ef_b, ws[D2:])
        eaexp = _dot(ea_t, q_ref[...])
        eaout_ref[...] = ea_t + _dot(t * eaexp, r_ref[...])

    return pl.pallas_call(
        body,
        grid=(ne // TE,),
        in_specs=[
            pl.BlockSpec((TE, FCI), lambda i: (i + ob, 0)),
            pl.BlockSpec((TE, EA), lambda i: (i + ea_ob, 0)),
            pl.BlockSpec((TE, D2), lambda i: (i, 0)),
            pl.BlockSpec((FCI, FCH), lambda i: (0, 0)),
            pl.BlockSpec((EA * FCH, D), lambda i: (0, 0)),
            pl.BlockSpec((D, EA * EA), lambda i: (0, 0)),
            pl.BlockSpec((EA, EA * FCH), lambda i: (0, 0)),
            pl.BlockSpec((EA, EA * EA), lambda i: (0, 0)),
            pl.BlockSpec((EA * EA, EA), lambda i: (0, 0)),
        ],
        out_specs=[
            pl.BlockSpec((TE, D), lambda i: (i, 0)),
            pl.BlockSpec((TE, EA), lambda i: (i, 0)),
        ],
        out_shape=[
            jax.ShapeDtypeStruct((ne, D), f32),
            jax.ShapeDtypeStruct((ne, EA), f32),
        ],
    )(es, ea, g, wfc1, w2, wsce, q64, q, r)


def _node_pre(x, wlin1, wsc):
    def body(x_ref, wlin1_ref, wsc_ref, nf_ref, nsc_ref):
        xt = x_ref[...]
        nf_ref[...] = _pack_bf16(_dot(xt, wlin1_ref[...]))
        nsc_ref[...] = _dot(xt, wsc_ref[...])

    return pl.pallas_call(
        body,
        grid=(N // TN,),
        in_specs=[
            pl.BlockSpec((TN, D), lambda i: (i, 0)),
            pl.BlockSpec((D, D), lambda i: (0, 0)),
            pl.BlockSpec((D, D), lambda i: (0, 0)),
        ],
        out_specs=[
            pl.BlockSpec((TN, D2), lambda i: (i, 0)),
            pl.BlockSpec((TN, D), lambda i: (i, 0)),
        ],
        out_shape=[
            jax.ShapeDtypeStruct((N, D2), f32),
            jax.ShapeDtypeStruct((N, D), f32),
        ],
    )(x, wlin1, wsc)


def _agg_sum(pa_ref, pb_ref):
    return ((pa_ref[0] + pa_ref[1]) + (pb_ref[0] + pb_ref[1]))


def _node_mid(aggpa, aggpb, nsc, wlin2, walpha, wlin1_2, wsc_2):
    def body(pa_ref, pb_ref, nsc_ref, wlin2_ref, walpha_ref, wlin1_2_ref,
             wsc_2_ref, nf2_ref, nsc2_ref):
        agg = _agg_sum(pa_ref, pb_ref)
        nco = _dot(agg, wlin2_ref[...])
        alpha = _dot(agg, walpha_ref[...])
        x2 = _fast_sin(nsc_ref[...] + alpha * nco)
        nf2_ref[...] = _pack_bf16(_dot(x2, wlin1_2_ref[...]))
        nsc2_ref[...] = _dot(x2, wsc_2_ref[...])

    return pl.pallas_call(
        body,
        grid=(N // TN,),
        in_specs=[
            pl.BlockSpec((NC, TN, D), lambda i: (0, i, 0)),
            pl.BlockSpec((NC, TN, D), lambda i: (0, i, 0)),
            pl.BlockSpec((TN, D), lambda i: (i, 0)),
            pl.BlockSpec((D, D), lambda i: (0, 0)),
            pl.BlockSpec((D, 1), lambda i: (0, 0)),
            pl.BlockSpec((D, D), lambda i: (0, 0)),
            pl.BlockSpec((D, D), lambda i: (0, 0)),
        ],
        out_specs=[
            pl.BlockSpec((TN, D2), lambda i: (i, 0)),
            pl.BlockSpec((TN, D), lambda i: (i, 0)),
        ],
        out_shape=[
            jax.ShapeDtypeStruct((N, D2), f32),
            jax.ShapeDtypeStruct((N, D), f32),
        ],
    )(aggpa, aggpb, nsc, wlin2, walpha, wlin1_2, wsc_2)


def _node_final(aggpa, aggpb, nsc, wlin2, walpha):
    def body(pa_ref, pb_ref, nsc_ref, wlin2_ref, walpha_ref, out_ref):
        agg = _agg_sum(pa_ref, pb_ref)
        nco = _dot(agg, wlin2_ref[...])
        alpha = _dot(agg, walpha_ref[...])
        out_ref[...] = nsc_ref[...] + alpha * nco

    return pl.pallas_call(
        body,
        grid=(N // TN,),
        in_specs=[
            pl.BlockSpec((NC, TN, D), lambda i: (0, i, 0)),
            pl.BlockSpec((NC, TN, D), lambda i: (0, i, 0)),
            pl.BlockSpec((TN, D), lambda i: (i, 0)),
            pl.BlockSpec((D, D), lambda i: (0, 0)),
            pl.BlockSpec((D, 1), lambda i: (0, 0)),
        ],
        out_specs=pl.BlockSpec((TN, D), lambda i: (i, 0)),
        out_shape=jax.ShapeDtypeStruct((N, D), f32),
    )(aggpa, aggpb, nsc, wlin2, walpha)


def _prep(W_sc, W_lin1, W_fc1, W_fc2, W_lin2, W_alpha, W_sce):
    inv_d = 1.0 / np.sqrt(D)
    inv_agg = 1.0 / np.sqrt(32.0)
    wsc = W_sc[:, 0, :] * inv_d
    wlin1 = W_lin1[:, 0, :] * inv_d
    wfc1 = W_fc1 / np.sqrt(FCI)
    w2 = (W_fc2.reshape(FCH, D, EA).transpose(2, 0, 1)
          / (np.sqrt(FCH) * np.sqrt(EA))).reshape(EA * FCH, D)
    w2 = w2.astype(jnp.bfloat16)
    wlin2 = W_lin2[:, 0, :] * (inv_d * inv_agg)
    walpha = W_alpha[:, 0, :] * (inv_d * inv_agg)
    wsce = W_sce.reshape(D, EA * EA) / (np.sqrt(D * EA) * np.sqrt(32.0))
    return wsc, wlin1, wfc1, w2, wlin2, walpha, wsce


def kernel(node_features, node_attr, edge_src, edge_dst, edge_attr,
           edge_scalars,
           l1_W_sc, l1_W_lin1, l1_W_fc1, l1_W_fc2, l1_W_lin2, l1_W_alpha,
           l1_W_sce,
           l2_W_sc, l2_W_lin1, l2_W_fc1, l2_W_fc2, l2_W_lin2, l2_W_alpha,
           l2_W_sce):
    del node_attr
    src = edge_src.astype(jnp.int32)
    dst = edge_dst.astype(jnp.int32)
    wsc1, wlin11, wfc11, w21, wlin21, walpha1, wsce1 = _prep(
        l1_W_sc, l1_W_lin1, l1_W_fc1, l1_W_fc2, l1_W_lin2, l1_W_alpha, l1_W_sce)
    wsc2, wlin12, wfc12, w22, wlin22, walpha2, wsce2 = _prep(
        l2_W_sc, l2_W_lin1, l2_W_fc1, l2_W_fc2, l2_W_lin2, l2_W_alpha, l2_W_sce)
    q = jnp.asarray(np.kron(np.eye(EA), np.ones((1, EA))), f32)
    q64 = jnp.asarray(np.kron(np.eye(EA), np.ones((1, FCH))), f32)
    r = jnp.asarray(np.kron(np.ones((EA, 1)), np.eye(EA)), f32)
    zeros_nd = jnp.zeros((N, D), f32)
    E2 = E // 2
    OB = E2 // TE

    nf1, nsc1 = _node_pre(node_features, wlin11, wsc1)
    g1a = _sc_gather(nf1, src, 0, E2)
    g1b = _sc_gather(nf1, src, E2, E2)
    ef1a, eo1a = _edge_pass(edge_scalars, edge_attr, g1a,
                            wfc11, w21, wsce1, q64, q, r, 0, 0)
    ef1b, eo1b = _edge_pass(edge_scalars, edge_attr, g1b,
                            wfc11, w21, wsce1, q64, q, r, OB, OB)
    p1a = _sc_scatter(ef1a, dst, zeros_nd, 0, E2)
    p1b = _sc_scatter(ef1b, dst, zeros_nd, E2, E2)
    nf2, nsc2 = _node_mid(p1a, p1b, nsc1, wlin21, walpha1, wlin12, wsc2)
    g2a = _sc_gather(nf2, src, 0, E2)
    g2b = _sc_gather(nf2, src, E2, E2)
    ef2a, ea2a = _edge_pass(edge_scalars, eo1a, g2a,
                            wfc12, w22, wsce2, q64, q, r, 0, 0)
    ef2b, ea2b = _edge_pass(edge_scalars, eo1b, g2b,
                            wfc12, w22, wsce2, q64, q, r, OB, 0)
    p2a = _sc_scatter(ef2a, dst, zeros_nd, 0, E2)
    p2b = _sc_scatter(ef2b, dst, zeros_nd, E2, E2)
    x_out = _node_final(p2a, p2b, nsc2, wlin22, walpha2)
    ea_out = jnp.concatenate([ea2a, ea2b], axis=0)
    return (x_out, ea_out)


# speedup vs baseline: 1.1603x; 1.1603x over previous
import jax
import jax.numpy as jnp
import numpy as np
from jax import lax
from jax.experimental import pallas as pl
from jax.experimental.pallas import tpu as pltpu
from jax.experimental.pallas import tpu_sc as plsc

N, E, D, EA, FCI, FCH = 10000, 320000, 128, 4, 8, 64
NC, NS = 2, 16
NW = NC * NS
C = 128
TE = 8000
TN = 2000
GKG = 3
GKS = 3

f32 = jnp.float32


def _sc_mesh():
    return plsc.VectorSubcoreMesh(core_axis_name="c", subcore_axis_name="s",
                                  num_cores=NC, num_subcores=NS)


def _sc_gather(nf, src, e0, ne):
    nch = ne // C
    cpw = nch // NW
    rem = nch - cpw * NW
    nss = cpw // (2 * GKG)
    tail = cpw - nss * 2 * GKG

    def body(nf_hbm, src_hbm, out_hbm, idx_v, rows_v, semg, sems_a, sems_b):
        wid = lax.axis_index("s") * NC + lax.axis_index("c")
        base = wid * (cpw * C)
        pltpu.sync_copy(src_hbm.at[pl.ds(e0 + base, cpw * C)], idx_v)

        def buf(k):
            return rows_v.at[pl.ds(k * C, C)]

        def gather(lc, k):
            pltpu.async_copy(nf_hbm.at[idx_v.at[pl.ds(lc * C, C)]],
                             buf(k), semg)

        def gdrain(k):
            pltpu.make_async_copy(nf_hbm.at[pl.ds(0, C)], buf(k), semg).wait()

        def store(lc, k, sem):
            pltpu.async_copy(buf(k), out_hbm.at[pl.ds(base + lc * C, C)], sem)

        def sdrain(k, sem):
            pltpu.make_async_copy(buf(k), out_hbm.at[pl.ds(base, C)],
                                  sem).wait()

        def half(s, lc0, k0, sem):
            @pl.when(s > 0)
            def _():
                for k in range(GKG):
                    sdrain(k0 + k, sem)
            for k in range(GKG):
                gather(lc0 + k, k0 + k)
            for k in range(GKG):
                gdrain(k0 + k)
            for k in range(GKG):
                store(lc0 + k, k0 + k, sem)

        def step(s, carry):
            lc0 = s * 2 * GKG
            half(s, lc0, 0, sems_a)
            half(s, lc0 + GKG, GKG, sems_b)
            return carry

        lax.fori_loop(0, nss, step, 0)
        for k in range(GKG):
            sdrain(k, sems_a)
            sdrain(GKG + k, sems_b)
        for t in range(tail):
            lc = nss * 2 * GKG + t
            gather(lc, 0)
            gdrain(0)
            pltpu.sync_copy(buf(0), out_hbm.at[pl.ds(base + lc * C, C)])

        if rem:
            @pl.when(wid < rem)
            def _():
                off = (NW * cpw + wid) * C
                pltpu.sync_copy(src_hbm.at[pl.ds(e0 + off, C)],
                                idx_v.at[pl.ds(0, C)])
                gather(0, 0)
                gdrain(0)
                pltpu.sync_copy(buf(0), out_hbm.at[pl.ds(off, C)])

    return pl.kernel(
        body,
        out_type=jax.ShapeDtypeStruct((ne, D), f32),
        mesh=_sc_mesh(),
        scratch_types=[
            pltpu.VMEM((cpw * C,), jnp.int32),
            pltpu.VMEM((2 * GKG * C, D), f32),
            pltpu.SemaphoreType.DMA,
            pltpu.SemaphoreType.DMA,
            pltpu.SemaphoreType.DMA,
        ],
    )(nf, src)


def _sc_scatter(ef, dst, zeros_nd, e0, ne):
    rps = 1000
    nch = ne // C
    cpw = nch // NW
    rem = nch - cpw * NW
    nss = cpw // GKS
    tail = cpw - nss * GKS

    def body(ef_hbm, dst_hbm, z_hbm, agg_hbm, rows_v,
             i0, i1, i2, agg_sp, seml, semi, semc0, semc1, semc2):
        ibufs = (i0, i1, i2)
        semcs = (semc0, semc1, semc2)
        cid = lax.axis_index("c")
        sid = lax.axis_index("s")
        wid = sid * NC + cid

        @pl.when(sid < N // rps)
        def _():
            pltpu.sync_copy(z_hbm.at[pl.ds(sid * rps, rps)],
                            agg_sp.at[pl.ds(sid * rps, rps)])

        plsc.subcore_barrier()
        base = wid * (cpw * C)

        def buf(k):
            return rows_v.at[pl.ds(k * C, C)]

        def load(lc, k):
            pltpu.async_copy(ef_hbm.at[pl.ds(base + lc * C, C)], buf(k), seml)

        def ldrain(k):
            pltpu.make_async_copy(ef_hbm.at[pl.ds(0, C)], buf(k), seml).wait()

        def scat(k):
            pltpu.async_copy(buf(k), agg_sp.at[ibufs[k]], semcs[k], add=True)

        def cdrain(k):
            pltpu.make_async_copy(buf(k), agg_sp.at[pl.ds(0, C)],
                                  semcs[k]).wait()

        def step(s, carry):
            lc0 = s * GKS
            for k in range(GKS):
                @pl.when(s > 0)
                def _(k=k):
                    cdrain(k)
                pltpu.async_copy(dst_hbm.at[pl.ds(e0 + base + (lc0 + k) * C,
                                                  C)],
                                 ibufs[k], semi)
                load(lc0 + k, k)
            for k in range(GKS):
                pltpu.make_async_copy(dst_hbm.at[pl.ds(0, C)],
                                      ibufs[k], semi).wait()
                ldrain(k)
                scat(k)
            return carry

        lax.fori_loop(0, nss, step, 0)
        for k in range(GKS):
            cdrain(k)
        for t in range(tail):
            lc = nss * GKS + t
            pltpu.sync_copy(dst_hbm.at[pl.ds(e0 + base + lc * C, C)], i0)
            pltpu.sync_copy(ef_hbm.at[pl.ds(base + lc * C, C)], buf(0))
            pltpu.sync_copy(buf(0), agg_sp.at[i0], add=True)

        if rem:
            @pl.when(wid < rem)
            def _():
                off = (NW * cpw + wid) * C
                pltpu.sync_copy(dst_hbm.at[pl.ds(e0 + off, C)], i0)
                pltpu.sync_copy(ef_hbm.at[pl.ds(off, C)], buf(0))
                pltpu.sync_copy(buf(0), agg_sp.at[i0], add=True)

        plsc.subcore_barrier()

        @pl.when(sid < N // rps)
        def _():
            pltpu.sync_copy(agg_sp.at[pl.ds(sid * rps, rps)],
                            agg_hbm.at[cid, pl.ds(sid * rps, rps)])

    return pl.kernel(
        body,
        out_type=jax.ShapeDtypeStruct((NC, N, D), f32),
        mesh=_sc_mesh(),
        scratch_types=[
            pltpu.VMEM((GKS * C, D), f32),
            pltpu.VMEM((C,), jnp.int32),
            pltpu.VMEM((C,), jnp.int32),
            pltpu.VMEM((C,), jnp.int32),
            pltpu.VMEM_SHARED((N, D), f32),
            pltpu.SemaphoreType.DMA,
            pltpu.SemaphoreType.DMA,
            pltpu.SemaphoreType.DMA,
            pltpu.SemaphoreType.DMA,
            pltpu.SemaphoreType.DMA,
        ],
    )(ef, dst, zeros_nd)


def _dot(a, b):
    return jnp.dot(a, b, preferred_element_type=f32)


_SIN_C = (9.9997941e-01, -1.6662442e-01, 8.3089975e-03,
          -1.9265186e-04, 2.1479674e-06)


def _fast_sin(x):
    two_pi = np.float32(2.0 * np.pi)
    q = jnp.round(x * np.float32(1.0 / (2.0 * np.pi)))
    r = x - q * two_pi
    r2 = r * r
    acc = jnp.full_like(x, np.float32(_SIN_C[4]))
    for k in (3, 2, 1, 0):
        acc = acc * r2 + np.float32(_SIN_C[k])
    return r * acc


def _edge_pass(es, ea, g, wfc1, w2, wsce, q64, q, r, ob, ea_ob):
    ne = g.shape[0]

    def body(es_ref, ea_ref, g_ref, wfc1_ref, w2_ref, wsce_ref, q64_ref,
             q_ref, r_ref, ef_ref, eaout_ref):
        h = _fast_sin(_dot(es_ref[...], wfc1_ref[...]))
        ea_t = ea_ref[...]
        eaexp64 = _dot(ea_t, q64_ref[...])
        h4 = (jnp.concatenate([h] * EA, axis=1) * eaexp64).astype(jnp.bfloat16)
        wv = _dot(h4, w2_ref[...])
        ef = g_ref[...] * wv
        ef_ref[...] = ef
        t = _dot(ef, wsce_ref[...])
        eaexp = _dot(ea_t, q_ref[...])
        eaout_ref[...] = ea_t + _dot(t * eaexp, r_ref[...])

    return pl.pallas_call(
        body,
        grid=(ne // TE,),
        in_specs=[
            pl.BlockSpec((TE, FCI), lambda i: (i + ob, 0)),
            pl.BlockSpec((TE, EA), lambda i: (i + ea_ob, 0)),
            pl.BlockSpec((TE, D), lambda i: (i, 0)),
            pl.BlockSpec((FCI, FCH), lambda i: (0, 0)),
            pl.BlockSpec((EA * FCH, D), lambda i: (0, 0)),
            pl.BlockSpec((D, EA * EA), lambda i: (0, 0)),
            pl.BlockSpec((EA, EA * FCH), lambda i: (0, 0)),
            pl.BlockSpec((EA, EA * EA), lambda i: (0, 0)),
            pl.BlockSpec((EA * EA, EA), lambda i: (0, 0)),
        ],
        out_specs=[
            pl.BlockSpec((TE, D), lambda i: (i, 0)),
            pl.BlockSpec((TE, EA), lambda i: (i, 0)),
        ],
        out_shape=[
            jax.ShapeDtypeStruct((ne, D), f32),
            jax.ShapeDtypeStruct((ne, EA), f32),
        ],
    )(es, ea, g, wfc1, w2, wsce, q64, q, r)


def _node_pre(x, wlin1, wsc):
    def body(x_ref, wlin1_ref, wsc_ref, nf_ref, nsc_ref):
        xt = x_ref[...]
        nf_ref[...] = _dot(xt, wlin1_ref[...])
        nsc_ref[...] = _dot(xt, wsc_ref[...])

    return pl.pallas_call(
        body,
        grid=(N // TN,),
        in_specs=[
            pl.BlockSpec((TN, D), lambda i: (i, 0)),
            pl.BlockSpec((D, D), lambda i: (0, 0)),
            pl.BlockSpec((D, D), lambda i: (0, 0)),
        ],
        out_specs=[
            pl.BlockSpec((TN, D), lambda i: (i, 0)),
            pl.BlockSpec((TN, D), lambda i: (i, 0)),
        ],
        out_shape=[
            jax.ShapeDtypeStruct((N, D), f32),
            jax.ShapeDtypeStruct((N, D), f32),
        ],
    )(x, wlin1, wsc)


def _agg_sum(pa_ref, pb_ref):
    return ((pa_ref[0] + pa_ref[1]) + (pb_ref[0] + pb_ref[1]))


def _node_mid(aggpa, aggpb, nsc, wlin2, walpha, wlin1_2, wsc_2):
    def body(pa_ref, pb_ref, nsc_ref, wlin2_ref, walpha_ref, wlin1_2_ref,
             wsc_2_ref, nf2_ref, nsc2_ref):
        agg = _agg_sum(pa_ref, pb_ref)
        nco = _dot(agg, wlin2_ref[...])
        alpha = _dot(agg, walpha_ref[...])
        x2 = _fast_sin(nsc_ref[...] + alpha * nco)
        nf2_ref[...] = _dot(x2, wlin1_2_ref[...])
        nsc2_ref[...] = _dot(x2, wsc_2_ref[...])

    return pl.pallas_call(
        body,
        grid=(N // TN,),
        in_specs=[
            pl.BlockSpec((NC, TN, D), lambda i: (0, i, 0)),
            pl.BlockSpec((NC, TN, D), lambda i: (0, i, 0)),
            pl.BlockSpec((TN, D), lambda i: (i, 0)),
            pl.BlockSpec((D, D), lambda i: (0, 0)),
            pl.BlockSpec((D, 1), lambda i: (0, 0)),
            pl.BlockSpec((D, D), lambda i: (0, 0)),
            pl.BlockSpec((D, D), lambda i: (0, 0)),
        ],
        out_specs=[
            pl.BlockSpec((TN, D), lambda i: (i, 0)),
            pl.BlockSpec((TN, D), lambda i: (i, 0)),
        ],
        out_shape=[
            jax.ShapeDtypeStruct((N, D), f32),
            jax.ShapeDtypeStruct((N, D), f32),
        ],
    )(aggpa, aggpb, nsc, wlin2, walpha, wlin1_2, wsc_2)


def _node_final(aggpa, aggpb, nsc, wlin2, walpha):
    def body(pa_ref, pb_ref, nsc_ref, wlin2_ref, walpha_ref, out_ref):
        agg = _agg_sum(pa_ref, pb_ref)
        nco = _dot(agg, wlin2_ref[...])
        alpha = _dot(agg, walpha_ref[...])
        out_ref[...] = nsc_ref[...] + alpha * nco

    return pl.pallas_call(
        body,
        grid=(N // TN,),
        in_specs=[
            pl.BlockSpec((NC, TN, D), lambda i: (0, i, 0)),
            pl.BlockSpec((NC, TN, D), lambda i: (0, i, 0)),
            pl.BlockSpec((TN, D), lambda i: (i, 0)),
            pl.BlockSpec((D, D), lambda i: (0, 0)),
            pl.BlockSpec((D, 1), lambda i: (0, 0)),
        ],
        out_specs=pl.BlockSpec((TN, D), lambda i: (i, 0)),
        out_shape=jax.ShapeDtypeStruct((N, D), f32),
    )(aggpa, aggpb, nsc, wlin2, walpha)


def _prep(W_sc, W_lin1, W_fc1, W_fc2, W_lin2, W_alpha, W_sce):
    inv_d = 1.0 / np.sqrt(D)
    inv_agg = 1.0 / np.sqrt(32.0)
    wsc = W_sc[:, 0, :] * inv_d
    wlin1 = W_lin1[:, 0, :] * inv_d
    wfc1 = W_fc1 / np.sqrt(FCI)
    w2 = (W_fc2.reshape(FCH, D, EA).transpose(2, 0, 1)
          / (np.sqrt(FCH) * np.sqrt(EA))).reshape(EA * FCH, D)
    w2 = w2.astype(jnp.bfloat16)
    wlin2 = W_lin2[:, 0, :] * (inv_d * inv_agg)
    walpha = W_alpha[:, 0, :] * (inv_d * inv_agg)
    wsce = W_sce.reshape(D, EA * EA) / (np.sqrt(D * EA) * np.sqrt(32.0))
    return wsc, wlin1, wfc1, w2, wlin2, walpha, wsce


def kernel(node_features, node_attr, edge_src, edge_dst, edge_attr,
           edge_scalars,
           l1_W_sc, l1_W_lin1, l1_W_fc1, l1_W_fc2, l1_W_lin2, l1_W_alpha,
           l1_W_sce,
           l2_W_sc, l2_W_lin1, l2_W_fc1, l2_W_fc2, l2_W_lin2, l2_W_alpha,
           l2_W_sce):
    del node_attr
    src = edge_src.astype(jnp.int32)
    dst = edge_dst.astype(jnp.int32)
    wsc1, wlin11, wfc11, w21, wlin21, walpha1, wsce1 = _prep(
        l1_W_sc, l1_W_lin1, l1_W_fc1, l1_W_fc2, l1_W_lin2, l1_W_alpha, l1_W_sce)
    wsc2, wlin12, wfc12, w22, wlin22, walpha2, wsce2 = _prep(
        l2_W_sc, l2_W_lin1, l2_W_fc1, l2_W_fc2, l2_W_lin2, l2_W_alpha, l2_W_sce)
    q = jnp.asarray(np.kron(np.eye(EA), np.ones((1, EA))), f32)
    q64 = jnp.asarray(np.kron(np.eye(EA), np.ones((1, FCH))), f32)
    r = jnp.asarray(np.kron(np.ones((EA, 1)), np.eye(EA)), f32)
    zeros_nd = jnp.zeros((N, D), f32)
    E2 = E // 2
    OB = E2 // TE

    nf1, nsc1 = _node_pre(node_features, wlin11, wsc1)
    g1a = _sc_gather(nf1, src, 0, E2)
    g1b = _sc_gather(nf1, src, E2, E2)
    ef1a, eo1a = _edge_pass(edge_scalars, edge_attr, g1a,
                            wfc11, w21, wsce1, q64, q, r, 0, 0)
    ef1b, eo1b = _edge_pass(edge_scalars, edge_attr, g1b,
                            wfc11, w21, wsce1, q64, q, r, OB, OB)
    p1a = _sc_scatter(ef1a, dst, zeros_nd, 0, E2)
    p1b = _sc_scatter(ef1b, dst, zeros_nd, E2, E2)
    nf2, nsc2 = _node_mid(p1a, p1b, nsc1, wlin21, walpha1, wlin12, wsc2)
    g2a = _sc_gather(nf2, src, 0, E2)
    g2b = _sc_gather(nf2, src, E2, E2)
    ef2a, ea2a = _edge_pass(edge_scalars, eo1a, g2a,
                            wfc12, w22, wsce2, q64, q, r, 0, 0)
    ef2b, ea2b = _edge_pass(edge_scalars, eo1b, g2b,
                            wfc12, w22, wsce2, q64, q, r, OB, 0)
    p2a = _sc_scatter(ef2a, dst, zeros_nd, 0, E2)
    p2b = _sc_scatter(ef2b, dst, zeros_nd, E2, E2)
    x_out = _node_final(p2a, p2b, nsc2, wlin22, walpha2)
    ea_out = jnp.concatenate([ea2a, ea2b], axis=0)
    return (x_out, ea_out)
